# baseline (speedup 1.0000x reference)
import jax
import jax.numpy as jnp
from jax.experimental import pallas as pl
from jax.experimental.pallas import tpu as pltpu
from jax.experimental.pallas import tpu_sc as plsc

_REG = 1e-5
_GATHER_WINDOW = 128
_TC_CHUNK = 2048


_REPACK_BLK = 160


def _sc_repack(table):
    rows, emb = table.shape
    half_blk = _REPACK_BLK // 2
    mesh = plsc.VectorSubcoreMesh(core_axis_name="core", subcore_axis_name="subcore")

    @pl.kernel(
        out_type=jax.ShapeDtypeStruct((rows // 2, 2 * emb), table.dtype),
        mesh=mesh,
    )
    def repack_kernel(x_hbm, o_hbm):
        def body(x_vmem, o_vmem):
            @pl.loop(0, half_blk)
            def _(r):
                @pl.loop(0, emb, step=16)
                def _(c):
                    o_vmem[r, pl.ds(c, 16)] = x_vmem[2 * r, pl.ds(c, 16)]
                    o_vmem[r, pl.ds(emb + c, 16)] = x_vmem[2 * r + 1, pl.ds(c, 16)]

        pltpu.emit_pipeline(
            body,
            grid=(rows // _REPACK_BLK,),
            in_specs=[pl.BlockSpec((_REPACK_BLK, emb), index_map=lambda i: (i, 0))],
            out_specs=[pl.BlockSpec((half_blk, 2 * emb), index_map=lambda i: (i, 0))],
            core_axis_name=("core", "subcore"),
            dimension_semantics=(pltpu.PARALLEL,),
        )(x_hbm, o_hbm)

    return repack_kernel(table)


def _sc_gather(packed_table, idx):
    n = idx.shape[0]
    width = packed_table.shape[1]
    idx2 = idx.reshape(1, n)
    mesh = plsc.VectorSubcoreMesh(core_axis_name="core", subcore_axis_name="subcore")

    @pl.kernel(
        out_type=jax.ShapeDtypeStruct((n, width), packed_table.dtype),
        mesh=mesh,
    )
    def gather_kernel(x_hbm, i_hbm, o_hbm):
        def body(i_vmem, o_vmem):
            pltpu.sync_copy(x_hbm.at[i_vmem.at[0]], o_vmem)

        pltpu.emit_pipeline(
            body,
            grid=(n // _GATHER_WINDOW,),
            in_specs=[pl.BlockSpec((1, _GATHER_WINDOW), index_map=lambda i: (0, i))],
            out_specs=[pl.BlockSpec((_GATHER_WINDOW, width), index_map=lambda i: (i, 0))],
            core_axis_name=("core", "subcore"),
            dimension_semantics=(pltpu.PARALLEL,),
        )(i_hbm, o_hbm)

    return gather_kernel(packed_table, idx2)


def _tc_reduce(gathered, parity, batch):
    width = gathered.shape[2]
    half = width // 2
    n_steps = gathered.shape[1] // _TC_CHUNK

    def body(g_ref, par_ref, loss_ref, bpr_ref, emb_ref, acc_ref):
        i = pl.program_id(0)

        @pl.when(i == 0)
        def _():
            acc_ref[0] = 0.0
            acc_ref[1] = 0.0

        lane = jax.lax.broadcasted_iota(jnp.int32, (_TC_CHUNK, width), 1)
        lane_lo = lane < half

        def mirror(k):
            par = par_ref[k][:, None] != 0
            m = jnp.where(lane_lo != par, g_ref[k], 0.0)
            return m + pltpu.roll(m, half, 1)

        u = mirror(0)
        p = mirror(1)
        ng = mirror(2)
        d = 0.5 * jnp.sum(u * (p - ng), axis=1)
        acc_ref[0] += jnp.sum(jax.nn.log_sigmoid(d.reshape(-1, 128)))
        acc_ref[1] += 0.5 * (jnp.sum(u * u) + jnp.sum(p * p) + jnp.sum(ng * ng))

        @pl.when(i == n_steps - 1)
        def _():
            bpr = -acc_ref[0] / batch
            emb = _REG * acc_ref[1] / (2.0 * batch)
            bpr_ref[0, 0] = bpr
            emb_ref[0, 0] = emb
            loss_ref[0, 0] = bpr + emb

    out_shape = [jax.ShapeDtypeStruct((1, 1), jnp.float32)] * 3
    smem = pl.BlockSpec(memory_space=pltpu.SMEM)
    loss, bpr, emb = pl.pallas_call(
        body,
        grid=(n_steps,),
        in_specs=[
            pl.BlockSpec((3, _TC_CHUNK, width), lambda i: (0, i, 0)),
            pl.BlockSpec((3, _TC_CHUNK), lambda i: (0, i)),
        ],
        out_shape=out_shape,
        out_specs=[smem, smem, smem],
        scratch_shapes=[pltpu.SMEM((2,), jnp.float32)],
    )(gathered, parity)
    return loss[0, 0], bpr[0, 0], emb[0, 0]


def kernel(all_embed, u_id, pos_i_id, neg_i_id):
    batch = u_id.shape[0]
    emb = all_embed.shape[1]
    idx = jnp.concatenate([u_id, pos_i_id, neg_i_id]).astype(jnp.int32)
    packed = _sc_repack(all_embed)
    gathered = _sc_gather(packed, idx // 2)
    gathered = gathered.reshape(3, batch, 2 * emb)
    parity = (idx & 1).reshape(3, batch)
    loss, bpr, emb_loss = _tc_reduce(gathered, parity, float(batch))
    reward = jnp.float32(0.0)
    return (reward, loss, bpr, emb_loss)


# speedup vs baseline: 1.1895x; 1.1895x over previous
import jax
import jax.numpy as jnp
from jax.experimental import pallas as pl
from jax.experimental.pallas import tpu as pltpu
from jax.experimental.pallas import tpu_sc as plsc

_REG = 1e-5
_GATHER_WINDOW = 128
_TC_CHUNK = 2048


def _sc_gather(packed_table, idx):
    n = idx.shape[0]
    width = packed_table.shape[1]
    idx2 = idx.reshape(1, n)
    mesh = plsc.VectorSubcoreMesh(core_axis_name="core", subcore_axis_name="subcore")

    @pl.kernel(
        out_type=jax.ShapeDtypeStruct((n, width), packed_table.dtype),
        mesh=mesh,
    )
    def gather_kernel(x_hbm, i_hbm, o_hbm):
        def body(i_vmem, o_vmem):
            pltpu.sync_copy(x_hbm.at[i_vmem.at[0]], o_vmem)

        pltpu.emit_pipeline(
            body,
            grid=(n // _GATHER_WINDOW,),
            in_specs=[pl.BlockSpec((1, _GATHER_WINDOW), index_map=lambda i: (0, i))],
            out_specs=[pl.BlockSpec((_GATHER_WINDOW, width), index_map=lambda i: (i, 0))],
            core_axis_name=("core", "subcore"),
            dimension_semantics=(pltpu.PARALLEL,),
        )(i_hbm, o_hbm)

    return gather_kernel(packed_table, idx2)


def _tc_reduce(gathered, batch):
    width = gathered.shape[2]
    n_steps = gathered.shape[1] // _TC_CHUNK

    def body(g_ref, loss_ref, bpr_ref, emb_ref, acc_ref):
        i = pl.program_id(0)

        @pl.when(i == 0)
        def _():
            acc_ref[0] = 0.0
            acc_ref[1] = 0.0

        g = g_ref[...]
        d = 0.5 * jnp.sum(g[0] * (g[1] - g[2]), axis=1)
        acc_ref[0] += jnp.sum(jax.nn.log_sigmoid(d.reshape(-1, 128)))
        acc_ref[1] += 0.5 * jnp.sum(g * g)

        @pl.when(i == n_steps - 1)
        def _():
            bpr = -acc_ref[0] / batch
            emb = _REG * acc_ref[1] / (2.0 * batch)
            bpr_ref[0, 0] = bpr
            emb_ref[0, 0] = emb
            loss_ref[0, 0] = bpr + emb

    out_shape = [jax.ShapeDtypeStruct((1, 1), jnp.float32)] * 3
    smem = pl.BlockSpec(memory_space=pltpu.SMEM)
    loss, bpr, emb = pl.pallas_call(
        body,
        grid=(n_steps,),
        in_specs=[pl.BlockSpec((3, _TC_CHUNK, width), lambda i: (0, i, 0))],
        out_shape=out_shape,
        out_specs=[smem, smem, smem],
        scratch_shapes=[pltpu.SMEM((2,), jnp.float32)],
    )(gathered)
    return loss[0, 0], bpr[0, 0], emb[0, 0]


def kernel(all_embed, u_id, pos_i_id, neg_i_id):
    batch = u_id.shape[0]
    emb = all_embed.shape[1]
    dup = jnp.concatenate([all_embed, all_embed], axis=1)
    idx = jnp.concatenate([u_id, pos_i_id, neg_i_id]).astype(jnp.int32)
    gathered = _sc_gather(dup, idx)
    gathered = gathered.reshape(3, batch, 2 * emb)
    loss, bpr, emb_loss = _tc_reduce(gathered, float(batch))
    reward = jnp.float32(0.0)
    return (reward, loss, bpr, emb_loss)


# speedup vs baseline: 1.2053x; 1.0132x over previous
import jax
import jax.numpy as jnp
from jax.experimental import pallas as pl
from jax.experimental.pallas import tpu as pltpu
from jax.experimental.pallas import tpu_sc as plsc

_REG = 1e-5
_GATHER_WINDOW = 256
_TC_CHUNK = 2048


def _sc_gather(packed_table, idx):
    n = idx.shape[0]
    width = packed_table.shape[1]
    idx2 = idx.reshape(1, n)
    mesh = plsc.VectorSubcoreMesh(core_axis_name="core", subcore_axis_name="subcore")

    @pl.kernel(
        out_type=jax.ShapeDtypeStruct((n, width), packed_table.dtype),
        mesh=mesh,
    )
    def gather_kernel(x_hbm, i_hbm, o_hbm):
        def body(i_vmem, o_vmem):
            pltpu.sync_copy(x_hbm.at[i_vmem.at[0]], o_vmem)

        pltpu.emit_pipeline(
            body,
            grid=(n // _GATHER_WINDOW,),
            in_specs=[pl.BlockSpec((1, _GATHER_WINDOW), index_map=lambda i: (0, i))],
            out_specs=[pl.BlockSpec((_GATHER_WINDOW, width), index_map=lambda i: (i, 0))],
            core_axis_name=("core", "subcore"),
            dimension_semantics=(pltpu.PARALLEL,),
        )(i_hbm, o_hbm)

    return gather_kernel(packed_table, idx2)


def _tc_reduce(gathered, parity, batch):
    width = gathered.shape[2]
    half = width // 2
    n_steps = gathered.shape[1] // _TC_CHUNK

    def body(g_ref, par_ref, loss_ref, bpr_ref, emb_ref, acc_ref):
        i = pl.program_id(0)

        @pl.when(i == 0)
        def _():
            acc_ref[0] = 0.0
            acc_ref[1] = 0.0

        lane = jax.lax.broadcasted_iota(jnp.int32, (_TC_CHUNK, width), 1)
        lane_lo = lane < half

        def mirror(k):
            par = par_ref[k][:, None] != 0
            m = jnp.where(lane_lo != par, g_ref[k], 0.0)
            return m + pltpu.roll(m, half, 1)

        u = mirror(0)
        p = mirror(1)
        ng = mirror(2)
        d = 0.5 * jnp.sum(u * (p - ng), axis=1)
        acc_ref[0] += jnp.sum(jax.nn.log_sigmoid(d.reshape(-1, 128)))
        acc_ref[1] += 0.5 * (jnp.sum(u * u) + jnp.sum(p * p) + jnp.sum(ng * ng))

        @pl.when(i == n_steps - 1)
        def _():
            bpr = -acc_ref[0] / batch
            emb = _REG * acc_ref[1] / (2.0 * batch)
            bpr_ref[0, 0] = bpr
            emb_ref[0, 0] = emb
            loss_ref[0, 0] = bpr + emb

    out_shape = [jax.ShapeDtypeStruct((1, 1), jnp.float32)] * 3
    smem = pl.BlockSpec(memory_space=pltpu.SMEM)
    loss, bpr, emb = pl.pallas_call(
        body,
        grid=(n_steps,),
        in_specs=[
            pl.BlockSpec((3, _TC_CHUNK, width), lambda i: (0, i, 0)),
            pl.BlockSpec((3, _TC_CHUNK), lambda i: (0, i)),
        ],
        out_shape=out_shape,
        out_specs=[smem, smem, smem],
        scratch_shapes=[pltpu.SMEM((2,), jnp.float32)],
    )(gathered, parity)
    return loss[0, 0], bpr[0, 0], emb[0, 0]


def kernel(all_embed, u_id, pos_i_id, neg_i_id):
    batch = u_id.shape[0]
    n_rows, emb = all_embed.shape
    packed = all_embed.reshape(n_rows // 2, 2 * emb)
    idx = jnp.concatenate([u_id, pos_i_id, neg_i_id]).astype(jnp.int32)
    gathered = _sc_gather(packed, idx // 2)
    gathered = gathered.reshape(3, batch, 2 * emb)
    parity = (idx & 1).reshape(3, batch)
    loss, bpr, emb_loss = _tc_reduce(gathered, parity, float(batch))
    reward = jnp.float32(0.0)
    return (reward, loss, bpr, emb_loss)
